# baseline (speedup 1.0000x reference)
import functools

import jax
import jax.numpy as jnp
from jax import lax
from jax.experimental import pallas as pl
from jax.experimental.pallas import tpu as pltpu
from jax.experimental.pallas import tpu_sc as plsc

N = 10000
E = 320000
D_IN = 128
H1 = 256
H2 = 32
N_CLASS = 64

NC = 2
NS = 16
NW = NC * NS
EPW = E // NW
EPW_PAD = 10240
NPAD = 10240
RPT = NPAD // NS


def _make_spmm(D, C, R, P):
  chunks = EPW_PAD // C
  assert chunks % R == 0 and C % 16 == 0

  mesh = plsc.VectorSubcoreMesh(
      core_axis_name="c", subcore_axis_name="s", num_cores=NC, num_subcores=NS)

  @functools.partial(
      pl.kernel,
      out_type=jax.ShapeDtypeStruct((P, NC * NPAD, D), jnp.float32),
      mesh=mesh,
      scratch_types=[
          pltpu.VMEM((chunks, C), jnp.int32),
          pltpu.VMEM((chunks, C), jnp.int32),
          pltpu.VMEM((chunks, C), jnp.float32),
      ] + [pltpu.VMEM((C, D), jnp.float32) for _ in range(R)]
        + [pltpu.VMEM_SHARED((NPAD, D), jnp.float32),
           pltpu.VMEM_SHARED((NPAD, D), jnp.float32)]
        + [pltpu.SemaphoreType.DMA for _ in range(R)],
      compiler_params=pltpu.CompilerParams(use_tc_tiling_on_sc=False),
  )
  def spmm(x_hbm, src_hbm, dst_hbm, w_hbm, zeros_hbm, out_hbm,
           src_all, dst_all, w_all, *ring):
    rows = ring[:R]
    acc = ring[R]
    x_src = ring[R + 1]
    gsem = ring[R + 2:]
    c = lax.axis_index("c")
    s = lax.axis_index("s")
    rbase = pl.multiple_of(s * RPT, 8)
    obase = pl.multiple_of(c * NPAD + s * RPT, 8)
    gbase = (c * NS + s) * chunks
    pltpu.sync_copy(src_hbm.at[pl.ds(gbase, chunks)], src_all)
    pltpu.sync_copy(dst_hbm.at[pl.ds(gbase, chunks)], dst_all)
    pltpu.sync_copy(w_hbm.at[pl.ds(gbase, chunks)], w_all)

    for p in range(P):
      pltpu.sync_copy(zeros_hbm.at[pl.ds(rbase, RPT)],
                      acc.at[pl.ds(rbase, RPT)])
      pltpu.sync_copy(x_hbm.at[p, pl.ds(rbase, RPT)],
                      x_src.at[pl.ds(rbase, RPT)])
      plsc.subcore_barrier()

      for j in range(R - 1):
        pltpu.async_copy(x_src.at[src_all.at[j]], rows[j], gsem[j])

      def body(kR, carry):
        for b in range(R):
          k = kR * R + b
          nb = (b + R - 1) % R

          @pl.when(k + R - 1 < chunks)
          def _prefetch():
            pltpu.async_copy(x_src.at[src_all.at[k + R - 1]], rows[nb],
                             gsem[nb])

          pltpu.make_async_copy(x_src.at[src_all.at[k]], rows[b],
                                gsem[b]).wait()

          def scale(g, carry2):
            wvec = w_all[k, pl.ds(pl.multiple_of(g * 16, 8), 16)]
            for l in range(16):
              wl = wvec[l]
              r = g * 16 + l
              for j in range(D // 16):
                sl = pl.ds(j * 16, 16)
                rows[b][r, sl] = rows[b][r, sl] * wl
            return carry2

          lax.fori_loop(0, C // 16, scale, 0)
          pltpu.sync_copy(rows[b], acc.at[dst_all.at[k]], add=True)
        return carry

      lax.fori_loop(0, chunks // R, body, 0)
      plsc.subcore_barrier()
      pltpu.sync_copy(acc.at[pl.ds(rbase, RPT)],
                      out_hbm.at[p, pl.ds(obase, RPT)])

  return spmm


_C128, _R128 = 64, 4
_C32, _R32 = 320, 4
_spmm128 = _make_spmm(64, _C128, _R128, 2)
_spmm32 = _make_spmm(H2, _C32, _R32, 1)


def _pad_edges(src, dst, w):
  pad = EPW_PAD - EPW
  src_p = jnp.pad(src.reshape(NW, EPW), ((0, 0), (0, pad)))
  dst_p = jnp.pad(dst.reshape(NW, EPW), ((0, 0), (0, pad)))
  w_p = jnp.pad(w.reshape(NW, EPW), ((0, 0), (0, pad)))
  return src_p.reshape(-1), dst_p.reshape(-1), w_p.reshape(-1)


def _fc1_body(p00_ref, p01_ref, p10_ref, p11_ref, w1a_ref, w1b_ref, b1_ref,
              w2_ref, z_ref):
  a0 = p00_ref[0] + p01_ref[0]
  a1 = p10_ref[0] + p11_ref[0]
  h = (jnp.dot(a0, w1a_ref[...], preferred_element_type=jnp.float32) +
       jnp.dot(a1, w1b_ref[...], preferred_element_type=jnp.float32))
  h = jnp.maximum(h + b1_ref[...], 0.0)
  z_ref[...] = jnp.dot(h, w2_ref[...], preferred_element_type=jnp.float32)


def _head_body(q0_ref, q1_ref, b2_ref, w3_ref, b3_ref, out_ref, t_ref):
  t = jnp.maximum(q0_ref[0] + q1_ref[0] + b2_ref[...], 0.0)
  x3 = jnp.dot(t, w3_ref[...], preferred_element_type=jnp.float32) + b3_ref[...]
  m = jnp.max(x3, axis=1, keepdims=True)
  lse = jnp.log(jnp.sum(jnp.exp(x3 - m), axis=1, keepdims=True)) + m
  out_ref[...] = x3 - lse
  t_ref[...] = t


_BM = 1024
_NBLK = NPAD // _BM


def _fc1(p, W1, b1, W2):
  W1a = W1[:64]
  W1b = W1[64:]
  return pl.pallas_call(
      _fc1_body,
      grid=(_NBLK,),
      in_specs=[
          pl.BlockSpec((1, _BM, 64), lambda i: (0, i, 0)),
          pl.BlockSpec((1, _BM, 64), lambda i: (0, i + _NBLK, 0)),
          pl.BlockSpec((1, _BM, 64), lambda i: (1, i, 0)),
          pl.BlockSpec((1, _BM, 64), lambda i: (1, i + _NBLK, 0)),
          pl.BlockSpec((64, H1), lambda i: (0, 0)),
          pl.BlockSpec((64, H1), lambda i: (0, 0)),
          pl.BlockSpec((1, H1), lambda i: (0, 0)),
          pl.BlockSpec((H1, H2), lambda i: (0, 0)),
      ],
      out_specs=pl.BlockSpec((_BM, H2), lambda i: (i, 0)),
      out_shape=jax.ShapeDtypeStruct((NPAD, H2), jnp.float32),
  )(p, p, p, p, W1a, W1b, b1.reshape(1, H1), W2)


def _head(q, b2, W3, b3):
  return pl.pallas_call(
      _head_body,
      grid=(_NBLK,),
      in_specs=[
          pl.BlockSpec((1, _BM, H2), lambda i: (0, i, 0)),
          pl.BlockSpec((1, _BM, H2), lambda i: (0, i + _NBLK, 0)),
          pl.BlockSpec((1, H2), lambda i: (0, 0)),
          pl.BlockSpec((H2, N_CLASS), lambda i: (0, 0)),
          pl.BlockSpec((1, N_CLASS), lambda i: (0, 0)),
      ],
      out_specs=[
          pl.BlockSpec((_BM, N_CLASS), lambda i: (i, 0)),
          pl.BlockSpec((_BM, H2), lambda i: (i, 0)),
      ],
      out_shape=[
          jax.ShapeDtypeStruct((N, N_CLASS), jnp.float32),
          jax.ShapeDtypeStruct((N, H2), jnp.float32),
      ],
  )(q, q, b2.reshape(1, H2), W3, b3.reshape(1, N_CLASS))


def kernel(x_in, edge_index, edge_weight, W1, b1, W2, b2, W3, b3):
  dst = edge_index[0]
  src = edge_index[1]
  src_p, dst_p, w_p = _pad_edges(src, dst, edge_weight)
  x_pad = jnp.pad(x_in, ((0, NPAD - N), (0, 0)))
  x2 = jnp.stack([x_pad[:, :64], x_pad[:, 64:]])
  src128 = src_p.reshape(-1, _C128)
  dst128 = dst_p.reshape(-1, _C128)
  w128 = w_p.reshape(-1, _C128)
  p = _spmm128(x2, src128, dst128, w128,
               jnp.zeros((NPAD, 64), jnp.float32))
  z = _fc1(p, W1, b1, W2)
  src32 = src_p.reshape(-1, _C32)
  dst32 = dst_p.reshape(-1, _C32)
  w32 = w_p.reshape(-1, _C32)
  q = _spmm32(z.reshape(1, NPAD, H2), src32, dst32, w32,
              jnp.zeros((NPAD, H2), jnp.float32))
  out, t = _head(q, b2, W3, b3)
  return (out, t)


# speedup vs baseline: 1.6966x; 1.6966x over previous
import functools

import jax
import jax.numpy as jnp
from jax import lax
from jax.experimental import pallas as pl
from jax.experimental.pallas import tpu as pltpu
from jax.experimental.pallas import tpu_sc as plsc

N = 10000
E = 320000
D_IN = 128
H1 = 256
H2 = 32
N_CLASS = 64

NC = 2
NS = 16
NW = NC * NS
EPW = E // NW
EPW_PAD = 10240
NPAD = 10240
RPT = NPAD // NS


def _make_spmm(D, C, R, P):
  chunks = EPW_PAD // C
  assert chunks % R == 0 and C % 16 == 0

  mesh = plsc.VectorSubcoreMesh(
      core_axis_name="c", subcore_axis_name="s", num_cores=NC, num_subcores=NS)

  @functools.partial(
      pl.kernel,
      out_type=jax.ShapeDtypeStruct((P, NC * NPAD, D), jnp.float32),
      mesh=mesh,
      scratch_types=[
          pltpu.VMEM((chunks, C), jnp.int32),
          pltpu.VMEM((chunks, C), jnp.int32),
          pltpu.VMEM((chunks, C), jnp.float32),
      ] + [pltpu.VMEM((C, D), jnp.float32) for _ in range(R)]
        + [pltpu.VMEM_SHARED((NPAD, D), jnp.float32),
           pltpu.VMEM_SHARED((NPAD, D), jnp.float32)]
        + [pltpu.SemaphoreType.DMA for _ in range(R)],
      compiler_params=pltpu.CompilerParams(use_tc_tiling_on_sc=False),
  )
  def spmm(x_hbm, src_hbm, dst_hbm, w_hbm, zeros_hbm, out_hbm,
           src_all, dst_all, w_all, *ring):
    rows = ring[:R]
    acc = ring[R]
    x_src = ring[R + 1]
    gsem = ring[R + 2:]
    c = lax.axis_index("c")
    s = lax.axis_index("s")
    rbase = pl.multiple_of(s * RPT, 8)
    obase = pl.multiple_of(c * NPAD + s * RPT, 8)
    gbase = (c * NS + s) * chunks
    pltpu.sync_copy(src_hbm.at[pl.ds(gbase, chunks)], src_all)
    pltpu.sync_copy(dst_hbm.at[pl.ds(gbase, chunks)], dst_all)
    pltpu.sync_copy(w_hbm.at[pl.ds(gbase, chunks)], w_all)

    for p in range(P):
      pltpu.sync_copy(zeros_hbm.at[pl.ds(rbase, RPT)],
                      acc.at[pl.ds(rbase, RPT)])
      pltpu.sync_copy(x_hbm.at[p, pl.ds(rbase, RPT)],
                      x_src.at[pl.ds(rbase, RPT)])
      plsc.subcore_barrier()

      for j in range(R - 1):
        pltpu.async_copy(x_src.at[src_all.at[j]], rows[j], gsem[j])

      def body(kR, carry):
        for b in range(R):
          k = kR * R + b
          nb = (b + R - 1) % R

          @pl.when(k + R - 1 < chunks)
          def _prefetch():
            pltpu.async_copy(x_src.at[src_all.at[k + R - 1]], rows[nb],
                             gsem[nb])

          pltpu.make_async_copy(x_src.at[src_all.at[k]], rows[b],
                                gsem[b]).wait()

          def scale(g, carry2):
            wvec = w_all[k, pl.ds(pl.multiple_of(g * 16, 8), 16)]
            for l in range(16):
              wl = wvec[l]
              r = g * 16 + l
              for j in range(D // 16):
                sl = pl.ds(j * 16, 16)
                rows[b][r, sl] = rows[b][r, sl] * wl
            return carry2

          lax.fori_loop(0, C // 16, scale, 0)
          pltpu.sync_copy(rows[b], acc.at[dst_all.at[k]], add=True)
        return carry

      lax.fori_loop(0, chunks // R, body, 0)
      plsc.subcore_barrier()
      pltpu.sync_copy(acc.at[pl.ds(rbase, RPT)],
                      out_hbm.at[p, pl.ds(obase, RPT)])

  return spmm


_C128, _R128 = 32, 8
_C32, _R32 = 160, 8
_spmm128 = _make_spmm(64, _C128, _R128, 2)
_spmm32 = _make_spmm(H2, _C32, _R32, 1)


def _pad_edges(src, dst, w):
  pad = EPW_PAD - EPW
  src_p = jnp.pad(src.reshape(NW, EPW), ((0, 0), (0, pad)))
  dst_p = jnp.pad(dst.reshape(NW, EPW), ((0, 0), (0, pad)))
  w_p = jnp.pad(w.reshape(NW, EPW), ((0, 0), (0, pad)))
  return src_p.reshape(-1), dst_p.reshape(-1), w_p.reshape(-1)


def _fc1_body(p00_ref, p01_ref, p10_ref, p11_ref, w1a_ref, w1b_ref, b1_ref,
              w2_ref, z_ref):
  a0 = p00_ref[0] + p01_ref[0]
  a1 = p10_ref[0] + p11_ref[0]
  h = (jnp.dot(a0, w1a_ref[...], preferred_element_type=jnp.float32) +
       jnp.dot(a1, w1b_ref[...], preferred_element_type=jnp.float32))
  h = jnp.maximum(h + b1_ref[...], 0.0)
  z_ref[...] = jnp.dot(h, w2_ref[...], preferred_element_type=jnp.float32)


def _head_body(q0_ref, q1_ref, b2_ref, w3_ref, b3_ref, out_ref, t_ref):
  t = jnp.maximum(q0_ref[0] + q1_ref[0] + b2_ref[...], 0.0)
  x3 = jnp.dot(t, w3_ref[...], preferred_element_type=jnp.float32) + b3_ref[...]
  m = jnp.max(x3, axis=1, keepdims=True)
  lse = jnp.log(jnp.sum(jnp.exp(x3 - m), axis=1, keepdims=True)) + m
  out_ref[...] = x3 - lse
  t_ref[...] = t


_BM = 1024
_NBLK = NPAD // _BM


def _fc1(p, W1, b1, W2):
  W1a = W1[:64]
  W1b = W1[64:]
  return pl.pallas_call(
      _fc1_body,
      grid=(_NBLK,),
      in_specs=[
          pl.BlockSpec((1, _BM, 64), lambda i: (0, i, 0)),
          pl.BlockSpec((1, _BM, 64), lambda i: (0, i + _NBLK, 0)),
          pl.BlockSpec((1, _BM, 64), lambda i: (1, i, 0)),
          pl.BlockSpec((1, _BM, 64), lambda i: (1, i + _NBLK, 0)),
          pl.BlockSpec((64, H1), lambda i: (0, 0)),
          pl.BlockSpec((64, H1), lambda i: (0, 0)),
          pl.BlockSpec((1, H1), lambda i: (0, 0)),
          pl.BlockSpec((H1, H2), lambda i: (0, 0)),
      ],
      out_specs=pl.BlockSpec((_BM, H2), lambda i: (i, 0)),
      out_shape=jax.ShapeDtypeStruct((NPAD, H2), jnp.float32),
  )(p, p, p, p, W1a, W1b, b1.reshape(1, H1), W2)


def _head(q, b2, W3, b3):
  return pl.pallas_call(
      _head_body,
      grid=(_NBLK,),
      in_specs=[
          pl.BlockSpec((1, _BM, H2), lambda i: (0, i, 0)),
          pl.BlockSpec((1, _BM, H2), lambda i: (0, i + _NBLK, 0)),
          pl.BlockSpec((1, H2), lambda i: (0, 0)),
          pl.BlockSpec((H2, N_CLASS), lambda i: (0, 0)),
          pl.BlockSpec((1, N_CLASS), lambda i: (0, 0)),
      ],
      out_specs=[
          pl.BlockSpec((_BM, N_CLASS), lambda i: (i, 0)),
          pl.BlockSpec((_BM, H2), lambda i: (i, 0)),
      ],
      out_shape=[
          jax.ShapeDtypeStruct((N, N_CLASS), jnp.float32),
          jax.ShapeDtypeStruct((N, H2), jnp.float32),
      ],
  )(q, q, b2.reshape(1, H2), W3, b3.reshape(1, N_CLASS))


def kernel(x_in, edge_index, edge_weight, W1, b1, W2, b2, W3, b3):
  dst = edge_index[0]
  src = edge_index[1]
  src_p, dst_p, w_p = _pad_edges(src, dst, edge_weight)
  x_pad = jnp.pad(x_in, ((0, NPAD - N), (0, 0)))
  x2 = jnp.stack([x_pad[:, :64], x_pad[:, 64:]])
  src128 = src_p.reshape(-1, _C128)
  dst128 = dst_p.reshape(-1, _C128)
  w128 = w_p.reshape(-1, _C128)
  p = _spmm128(x2, src128, dst128, w128,
               jnp.zeros((NPAD, 64), jnp.float32))
  z = _fc1(p, W1, b1, W2)
  src32 = src_p.reshape(-1, _C32)
  dst32 = dst_p.reshape(-1, _C32)
  w32 = w_p.reshape(-1, _C32)
  q = _spmm32(z.reshape(1, NPAD, H2), src32, dst32, w32,
              jnp.zeros((NPAD, H2), jnp.float32))
  out, t = _head(q, b2, W3, b3)
  return (out, t)
